# baseline (speedup 1.0000x reference)
import functools

import jax
import jax.numpy as jnp
from jax import lax
from jax.experimental import pallas as pl
from jax.experimental.pallas import tpu as pltpu
from jax.experimental.pallas import tpu_sc as plsc

MARGIN_SQ = 0.25

ROWS, COLS = 16384, 128
TOTAL = ROWS * COLS
NC, NS, L = 2, 16, 16
NW = NC * NS
PER_W = TOTAL // NW
CHUNK = 8192
NCHUNK = PER_W // CHUNK


def _sc_partials(pred_flat, real_flat):
    mesh = plsc.VectorSubcoreMesh(core_axis_name="c", subcore_axis_name="s")

    @functools.partial(
        pl.kernel,
        mesh=mesh,
        out_type=[
            jax.ShapeDtypeStruct((NW, L), jnp.float32),
            jax.ShapeDtypeStruct((NW, L), jnp.float32),
        ],
        scratch_types=[
            pltpu.VMEM((CHUNK,), jnp.float32),
            pltpu.VMEM((CHUNK,), jnp.float32),
            pltpu.VMEM((L,), jnp.float32),
            pltpu.VMEM((L,), jnp.float32),
        ],
    )
    def body(pred_hbm, real_hbm, sq_out, cnt_out, pbuf, rbuf, sq_v, cnt_v):
        wid = lax.axis_index("s") * NC + lax.axis_index("c")
        base = wid * PER_W

        def chunk_body(c, carry):
            acc_sq, acc_cnt = carry
            off = base + c * CHUNK
            pltpu.sync_copy(pred_hbm.at[pl.ds(off, CHUNK)], pbuf)
            pltpu.sync_copy(real_hbm.at[pl.ds(off, CHUNK)], rbuf)

            def vec_body(i, carry2):
                a_sq, a_cnt = carry2
                p = pbuf[pl.ds(i * L, L)]
                r = rbuf[pl.ds(i * L, L)]
                d = r - p
                sq = d * d
                m = sq > MARGIN_SQ
                a_sq = a_sq + jnp.where(m, sq, 0.0)
                a_cnt = a_cnt + jnp.where(m, 1.0, 0.0)
                return (a_sq, a_cnt)

            return lax.fori_loop(0, CHUNK // L, vec_body, (acc_sq, acc_cnt))

        zero = jnp.zeros((L,), jnp.float32)
        acc_sq, acc_cnt = lax.fori_loop(0, NCHUNK, chunk_body, (zero, zero))
        sq_v[...] = acc_sq
        cnt_v[...] = acc_cnt
        pltpu.sync_copy(sq_v, sq_out.at[wid])
        pltpu.sync_copy(cnt_v, cnt_out.at[wid])

    return body(pred_flat, real_flat)


def _combine_body(sq_ref, cnt_ref, out_ref):
    s = jnp.sum(sq_ref[...])
    n = jnp.sum(cnt_ref[...])
    out_ref[0, 0] = jnp.where(n > 0.0, s / jnp.maximum(n, 1.0), 0.0)


def _combine(sq, cnt):
    return pl.pallas_call(
        _combine_body,
        out_shape=jax.ShapeDtypeStruct((1, 1), jnp.float32),
        out_specs=pl.BlockSpec(memory_space=pltpu.SMEM),
    )(sq, cnt)


def kernel(pred, real):
    pred_flat = pred.reshape(TOTAL)
    real_flat = real.reshape(TOTAL)
    sq, cnt = _sc_partials(pred_flat, real_flat)
    out = _combine(sq, cnt)
    return out[0, 0]


# speedup vs baseline: 1.7027x; 1.7027x over previous
import functools

import jax
import jax.numpy as jnp
from jax import lax
from jax.experimental import pallas as pl
from jax.experimental.pallas import tpu as pltpu
from jax.experimental.pallas import tpu_sc as plsc

MARGIN_SQ = 0.25

ROWS, COLS = 16384, 128
TOTAL = ROWS * COLS
NC, NS, L = 2, 16, 16
NW = NC * NS
PER_W = TOTAL // NW
CHUNK = 16384
NCHUNK = PER_W // CHUNK
UNROLL = 8
NACC = 4


def _sc_partials(pred_flat, real_flat):
    mesh = plsc.VectorSubcoreMesh(core_axis_name="c", subcore_axis_name="s")

    @functools.partial(
        pl.kernel,
        mesh=mesh,
        out_type=[
            jax.ShapeDtypeStruct((NW, L), jnp.float32),
            jax.ShapeDtypeStruct((NW, L), jnp.float32),
        ],
        scratch_types=[
            pltpu.VMEM((2 * CHUNK,), jnp.float32),
            pltpu.VMEM((2 * CHUNK,), jnp.float32),
            pltpu.VMEM((L,), jnp.float32),
            pltpu.VMEM((L,), jnp.float32),
            pltpu.SemaphoreType.DMA,
            pltpu.SemaphoreType.DMA,
        ],
    )
    def body(pred_hbm, real_hbm, sq_out, cnt_out, pbuf, rbuf, sq_v, cnt_v,
             sem0, sem1):
        wid = lax.axis_index("s") * NC + lax.axis_index("c")
        base = wid * PER_W
        sems = (sem0, sem1)

        def start(c):
            b = c % 2
            off = base + c * CHUNK
            hp = pltpu.async_copy(
                pred_hbm.at[pl.ds(off, CHUNK)],
                pbuf.at[pl.ds(b * CHUNK, CHUNK)], sems[b])
            hr = pltpu.async_copy(
                real_hbm.at[pl.ds(off, CHUNK)],
                rbuf.at[pl.ds(b * CHUNK, CHUNK)], sems[b])
            return hp, hr

        handles = [None] * NCHUNK
        handles[0] = start(0)

        zf = jnp.zeros((L,), jnp.float32)
        accs = (zf,) * (2 * NACC)

        for c in range(NCHUNK):
            if c + 1 < NCHUNK:
                handles[c + 1] = start(c + 1)
            hp, hr = handles[c]
            hp.wait()
            hr.wait()
            vbase = (c % 2) * CHUNK

            def vec_body(i, acc, vbase=vbase):
                sqs = list(acc[:NACC])
                cnts = list(acc[NACC:])
                o = vbase + i * (L * UNROLL)
                for u in range(UNROLL):
                    p = pbuf[pl.ds(o + u * L, L)]
                    r = rbuf[pl.ds(o + u * L, L)]
                    d = r - p
                    sq = d * d
                    m = sq > MARGIN_SQ
                    a = u % NACC
                    sqs[a] = sqs[a] + jnp.where(m, sq, 0.0)
                    cnts[a] = cnts[a] + jnp.where(m, 1.0, 0.0)
                return tuple(sqs) + tuple(cnts)

            accs = lax.fori_loop(0, CHUNK // (L * UNROLL), vec_body, accs)

        acc_sq = accs[0]
        for a in range(1, NACC):
            acc_sq = acc_sq + accs[a]
        acc_cnt = accs[NACC]
        for a in range(1, NACC):
            acc_cnt = acc_cnt + accs[NACC + a]

        sq_v[...] = acc_sq
        cnt_v[...] = acc_cnt
        pltpu.sync_copy(sq_v, sq_out.at[wid])
        pltpu.sync_copy(cnt_v, cnt_out.at[wid])

    return body(pred_flat, real_flat)


def _combine_body(sq_ref, cnt_ref, out_ref):
    s = jnp.sum(sq_ref[...])
    n = jnp.sum(cnt_ref[...])
    out_ref[0, 0] = jnp.where(n > 0.0, s / jnp.maximum(n, 1.0), 0.0)


def _combine(sq, cnt):
    return pl.pallas_call(
        _combine_body,
        out_shape=jax.ShapeDtypeStruct((1, 1), jnp.float32),
        out_specs=pl.BlockSpec(memory_space=pltpu.SMEM),
    )(sq, cnt)


def kernel(pred, real):
    pred_flat = pred.reshape(TOTAL)
    real_flat = real.reshape(TOTAL)
    sq, cnt = _sc_partials(pred_flat, real_flat)
    out = _combine(sq, cnt)
    return out[0, 0]
